# Initial kernel scaffold; baseline (speedup 1.0000x reference)
import functools

import jax
import jax.numpy as jnp
from jax import lax
from jax.experimental import pallas as pl
from jax.experimental.pallas import tpu as pltpu
from jax.experimental.pallas import tpu_sc as plsc

N = 10000
D = 128
E = 320000
NC, NS, LANES = 2, 16, 16
NW = NC * NS
NPAD = 10240
EPAD = 327680
ROWS_T = NPAD // NS
EPW = EPAD // NW
IDXROWS = EPAD // 128
ROWS_PW = EPW // 128
NSUPER = ROWS_PW // 16

_MESH = dict(core_axis_name="c", subcore_axis_name="s",
             num_cores=NC, num_subcores=NS)


def _worker():
    return lax.axis_index("c") * NS + lax.axis_index("s")


def _deg_body(dst2d, out, deg_sh, idx_v, ones_v, zero_v):
    c = lax.axis_index("c")
    s = lax.axis_index("s")

    def fill(i, _):
        ones_v[pl.ds(i * 16, 16)] = jnp.full((16,), 1.0, jnp.float32)
        return 0
    lax.fori_loop(0, 128 // 16, fill, 0)

    def zfill(i, _):
        zero_v[pl.ds(i * 16, 16)] = jnp.zeros((16,), jnp.float32)
        return 0
    lax.fori_loop(0, ROWS_T // 16, zfill, 0)
    pltpu.sync_copy(zero_v, deg_sh.at[pl.ds(ROWS_T * s, ROWS_T)])
    plsc.subcore_barrier()

    w = _worker()
    row0 = w * ROWS_PW
    for b in range(NSUPER):
        pltpu.sync_copy(dst2d.at[pl.ds(row0 + b * 16, 16)], idx_v)
        for j in range(16):
            pltpu.sync_copy(ones_v, deg_sh.at[idx_v.at[j]], add=True)
    plsc.subcore_barrier()
    pltpu.sync_copy(deg_sh.at[pl.ds(ROWS_T * s, ROWS_T)],
                    out.at[c, pl.ds(ROWS_T * s, ROWS_T)])


def _deg_call(dst2d):
    return pl.kernel(
        _deg_body,
        out_type=jax.ShapeDtypeStruct((NC, NPAD), jnp.float32),
        mesh=plsc.VectorSubcoreMesh(**_MESH),
        scratch_types=[
            pltpu.VMEM_SHARED((NPAD,), jnp.float32),
            pltpu.VMEM((16, 128), jnp.int32),
            pltpu.VMEM((128,), jnp.float32),
            pltpu.VMEM((ROWS_T,), jnp.float32),
        ],
    )(dst2d)


def _agg_body(m_hbm, src2d, dst2d, out, acc_sh, sidx_v, didx_v, rows_v, sem):
    c = lax.axis_index("c")
    s = lax.axis_index("s")
    pltpu.sync_copy(m_hbm.at[pl.ds(ROWS_T * s, ROWS_T)],
                    acc_sh.at[pl.ds(ROWS_T * s, ROWS_T)])
    plsc.subcore_barrier()

    w = _worker()
    row0 = w * ROWS_PW
    for b in range(NSUPER):
        pltpu.sync_copy(src2d.at[pl.ds(row0 + b * 16, 16)], sidx_v)
        pltpu.sync_copy(dst2d.at[pl.ds(row0 + b * 16, 16)], didx_v)
        for j in range(16):
            pltpu.async_copy(m_hbm.at[sidx_v.at[j]], rows_v, sem).wait()
            pltpu.sync_copy(rows_v, acc_sh.at[didx_v.at[j]], add=True)
    plsc.subcore_barrier()
    pltpu.sync_copy(acc_sh.at[pl.ds(ROWS_T * s, ROWS_T)],
                    out.at[c, pl.ds(ROWS_T * s, ROWS_T)])


def _agg_call(m, src2d, dst2d):
    return pl.kernel(
        _agg_body,
        out_type=jax.ShapeDtypeStruct((NC, NPAD, D), jnp.float32),
        mesh=plsc.VectorSubcoreMesh(**_MESH),
        scratch_types=[
            pltpu.VMEM_SHARED((NPAD, D), jnp.float32),
            pltpu.VMEM((16, 128), jnp.int32),
            pltpu.VMEM((16, 128), jnp.int32),
            pltpu.VMEM((128, D), jnp.float32),
            pltpu.SemaphoreType.DMA,
        ],
    )(m, src2d, dst2d)


def _cls_body(u_hbm, v_hbm, src2d, dst2d, w2_hbm, b2_hbm, out,
              sidx_v, didx_v, U_v, V_v, w2_v, b2_v, res_v, sem):
    pltpu.sync_copy(w2_hbm, w2_v)
    pltpu.sync_copy(b2_hbm, b2_v)
    w = _worker()
    row0 = w * ROWS_PW
    for b in range(NSUPER):
        pltpu.sync_copy(src2d.at[pl.ds(row0 + b * 16, 16)], sidx_v)
        pltpu.sync_copy(dst2d.at[pl.ds(row0 + b * 16, 16)], didx_v)
        for j in range(16):
            pltpu.async_copy(u_hbm.at[sidx_v.at[j]], U_v, sem).wait()
            pltpu.async_copy(v_hbm.at[didx_v.at[j]], V_v, sem).wait()

            def edge(e, _):
                acc = jnp.zeros((16,), jnp.float32)
                for k in range(D // 16):
                    uu = U_v[e, pl.ds(16 * k, 16)]
                    vv = V_v[e, pl.ds(16 * k, 16)]
                    t = jnp.maximum(uu + vv, 0.0)
                    acc = acc + t * w2_v[pl.ds(16 * k, 16)]
                res_v[e] = jnp.sum(acc)
                return 0
            lax.fori_loop(0, 128, edge, 0)
            bv = b2_v[...]
            for i in range(128 // 16):
                t = res_v[pl.ds(16 * i, 16)] + bv
                res_v[pl.ds(16 * i, 16)] = 1.0 / (1.0 + jnp.exp(-t))
            pltpu.sync_copy(
                res_v, out.at[pl.ds((row0 + b * 16 + j) * 128, 128)])


def _cls_call(u, v, src2d, dst2d, w2, b2):
    return pl.kernel(
        _cls_body,
        out_type=jax.ShapeDtypeStruct((EPAD,), jnp.float32),
        mesh=plsc.VectorSubcoreMesh(**_MESH),
        scratch_types=[
            pltpu.VMEM((16, 128), jnp.int32),
            pltpu.VMEM((16, 128), jnp.int32),
            pltpu.VMEM((128, D), jnp.float32),
            pltpu.VMEM((128, D), jnp.float32),
            pltpu.VMEM((D,), jnp.float32),
            pltpu.VMEM((16,), jnp.float32),
            pltpu.VMEM((128,), jnp.float32),
            pltpu.SemaphoreType.DMA,
        ],
    )(u, v, src2d, dst2d, w2, b2)


BLK = 1280


def _prep_body(deg_ref, x_ref, w_ref, dinv_ref, m_ref):
    deg = deg_ref[0] + deg_ref[1] + 1.0
    dinv = lax.rsqrt(deg)
    h = jnp.dot(x_ref[...], w_ref[...], preferred_element_type=jnp.float32)
    dinv_ref[...] = dinv
    m_ref[...] = h * dinv


def _prep_call(degs, x, w):
    return pl.pallas_call(
        _prep_body,
        grid=(NPAD // BLK,),
        in_specs=[
            pl.BlockSpec((NC, BLK, 1), lambda i: (0, i, 0)),
            pl.BlockSpec((BLK, D), lambda i: (i, 0)),
            pl.BlockSpec((D, D), lambda i: (0, 0)),
        ],
        out_specs=[
            pl.BlockSpec((BLK, 1), lambda i: (i, 0)),
            pl.BlockSpec((BLK, D), lambda i: (i, 0)),
        ],
        out_shape=[
            jax.ShapeDtypeStruct((NPAD, 1), jnp.float32),
            jax.ShapeDtypeStruct((NPAD, D), jnp.float32),
        ],
    )(degs, x, w)


def _layer_body(acc_ref, mprev_ref, dinv_ref, b_ref, w_ref, mnext_ref):
    dinv = dinv_ref[...]
    z = jnp.maximum(
        dinv * (acc_ref[0] + acc_ref[1] - mprev_ref[...]) + b_ref[...], 0.0)
    mnext_ref[...] = jnp.dot(
        z, w_ref[...], preferred_element_type=jnp.float32) * dinv


def _layer_call(acc, mprev, dinv, b, w):
    return pl.pallas_call(
        _layer_body,
        grid=(NPAD // BLK,),
        in_specs=[
            pl.BlockSpec((NC, BLK, D), lambda i: (0, i, 0)),
            pl.BlockSpec((BLK, D), lambda i: (i, 0)),
            pl.BlockSpec((BLK, 1), lambda i: (i, 0)),
            pl.BlockSpec((1, D), lambda i: (0, 0)),
            pl.BlockSpec((D, D), lambda i: (0, 0)),
        ],
        out_specs=pl.BlockSpec((BLK, D), lambda i: (i, 0)),
        out_shape=jax.ShapeDtypeStruct((NPAD, D), jnp.float32),
    )(acc, mprev, dinv, b, w)


def _final_body(acc_ref, mprev_ref, dinv_ref, b_ref, wa_ref, wb_ref, bl1_ref,
                u_ref, v_ref):
    dinv = dinv_ref[...]
    z = jnp.maximum(
        dinv * (acc_ref[0] + acc_ref[1] - mprev_ref[...]) + b_ref[...], 0.0)
    u_ref[...] = jnp.dot(
        z, wa_ref[...], preferred_element_type=jnp.float32) + bl1_ref[...]
    v_ref[...] = jnp.dot(z, wb_ref[...], preferred_element_type=jnp.float32)


def _final_call(acc, mprev, dinv, b, wa, wb, bl1):
    return pl.pallas_call(
        _final_body,
        grid=(NPAD // BLK,),
        in_specs=[
            pl.BlockSpec((NC, BLK, D), lambda i: (0, i, 0)),
            pl.BlockSpec((BLK, D), lambda i: (i, 0)),
            pl.BlockSpec((BLK, 1), lambda i: (i, 0)),
            pl.BlockSpec((1, D), lambda i: (0, 0)),
            pl.BlockSpec((D, D), lambda i: (0, 0)),
            pl.BlockSpec((D, D), lambda i: (0, 0)),
            pl.BlockSpec((1, D), lambda i: (0, 0)),
        ],
        out_specs=[
            pl.BlockSpec((BLK, D), lambda i: (i, 0)),
            pl.BlockSpec((BLK, D), lambda i: (i, 0)),
        ],
        out_shape=[
            jax.ShapeDtypeStruct((NPAD, D), jnp.float32),
            jax.ShapeDtypeStruct((NPAD, D), jnp.float32),
        ],
    )(acc, mprev, dinv, b, wa, wb, bl1)


def kernel(x, edge_index, Wc0, bc0, Wc1, bc1, Wc2, bc2, Wl1, bl1, Wl2, bl2):
    ei = edge_index.astype(jnp.int32)
    pad = N + (jnp.arange(EPAD - E, dtype=jnp.int32) % (NPAD - N))
    src2d = jnp.concatenate([ei[0], pad]).reshape(IDXROWS, 128)
    dst2d = jnp.concatenate([ei[1], pad]).reshape(IDXROWS, 128)
    x_p = jnp.zeros((NPAD, D), jnp.float32).at[:N].set(x)

    degs = _deg_call(dst2d)
    dinv, m0 = _prep_call(degs.reshape(NC, NPAD, 1), x_p, Wc0)
    a0 = _agg_call(m0, src2d, dst2d)
    m1 = _layer_call(a0, m0, dinv, bc0.reshape(1, D), Wc1)
    a1 = _agg_call(m1, src2d, dst2d)
    m2 = _layer_call(a1, m1, dinv, bc1.reshape(1, D), Wc2)
    a2 = _agg_call(m2, src2d, dst2d)
    u, v = _final_call(a2, m2, dinv, bc2.reshape(1, D),
                       Wl1[:D], Wl1[D:], bl1.reshape(1, D))
    outv = _cls_call(u, v, src2d, dst2d, Wl2.reshape(D),
                     jnp.broadcast_to(bl2, (16,)))
    return outv[:E].reshape(E, 1)


# speedup vs baseline: 10.1942x; 10.1942x over previous
import functools

import jax
import jax.numpy as jnp
from jax import lax
from jax.experimental import pallas as pl
from jax.experimental.pallas import tpu as pltpu
from jax.experimental.pallas import tpu_sc as plsc

N = 10000
D = 128
E = 320000
NC, NS, LANES = 2, 16, 16
NW = NC * NS
NPAD = 10240
EPAD = 327680
ROWS_T = NPAD // NS
EPW = EPAD // NW
IDXROWS = EPAD // 128
ROWS_PW = EPW // 128
NSUPER = ROWS_PW // 16

_MESH = dict(core_axis_name="c", subcore_axis_name="s",
             num_cores=NC, num_subcores=NS)


def _worker():
    return lax.axis_index("c") * NS + lax.axis_index("s")


def _deg_body(dst2d, out, deg_sh, idx_v, ones_v, zero_v):
    c = lax.axis_index("c")
    s = lax.axis_index("s")

    @pl.loop(0, 128 // 16)
    def _fill(i):
        ones_v[pl.ds(i * 16, 16)] = jnp.full((16,), 1.0, jnp.float32)

    @pl.loop(0, ROWS_T // 16)
    def _zfill(i):
        zero_v[pl.ds(i * 16, 16)] = jnp.zeros((16,), jnp.float32)

    pltpu.sync_copy(zero_v, deg_sh.at[pl.ds(ROWS_T * s, ROWS_T)])
    plsc.subcore_barrier()

    w = _worker()
    row0 = w * ROWS_PW

    @pl.loop(0, NSUPER)
    def _bloop(b):
        pltpu.sync_copy(dst2d.at[pl.ds(row0 + b * 16, 16)], idx_v)

        @pl.loop(0, 16)
        def _jloop(j):
            pltpu.sync_copy(ones_v, deg_sh.at[idx_v.at[j]], add=True)

    plsc.subcore_barrier()
    pltpu.sync_copy(deg_sh.at[pl.ds(ROWS_T * s, ROWS_T)],
                    out.at[c, pl.ds(ROWS_T * s, ROWS_T)])


def _deg_call(dst2d):
    return pl.kernel(
        _deg_body,
        out_type=jax.ShapeDtypeStruct((NC, NPAD), jnp.float32),
        mesh=plsc.VectorSubcoreMesh(**_MESH),
        scratch_types=[
            pltpu.VMEM_SHARED((NPAD,), jnp.float32),
            pltpu.VMEM((16, 128), jnp.int32),
            pltpu.VMEM((128,), jnp.float32),
            pltpu.VMEM((ROWS_T,), jnp.float32),
        ],
    )(dst2d)


def _agg_body(m_hbm, src2d, dst2d, out, acc_sh, sidx_v, didx_v, rows_v, sem):
    c = lax.axis_index("c")
    s = lax.axis_index("s")
    pltpu.sync_copy(m_hbm.at[pl.ds(ROWS_T * s, ROWS_T)],
                    acc_sh.at[pl.ds(ROWS_T * s, ROWS_T)])
    plsc.subcore_barrier()

    w = _worker()
    row0 = w * ROWS_PW

    @pl.loop(0, NSUPER)
    def _bloop(b):
        pltpu.sync_copy(src2d.at[pl.ds(row0 + b * 16, 16)], sidx_v)
        pltpu.sync_copy(dst2d.at[pl.ds(row0 + b * 16, 16)], didx_v)

        @pl.loop(0, 16)
        def _jloop(j):
            pltpu.async_copy(m_hbm.at[sidx_v.at[j]], rows_v, sem).wait()
            pltpu.sync_copy(rows_v, acc_sh.at[didx_v.at[j]], add=True)

    plsc.subcore_barrier()
    pltpu.sync_copy(acc_sh.at[pl.ds(ROWS_T * s, ROWS_T)],
                    out.at[c, pl.ds(ROWS_T * s, ROWS_T)])


def _agg_call(m, src2d, dst2d):
    return pl.kernel(
        _agg_body,
        out_type=jax.ShapeDtypeStruct((NC, NPAD, D), jnp.float32),
        mesh=plsc.VectorSubcoreMesh(**_MESH),
        scratch_types=[
            pltpu.VMEM_SHARED((NPAD, D), jnp.float32),
            pltpu.VMEM((16, 128), jnp.int32),
            pltpu.VMEM((16, 128), jnp.int32),
            pltpu.VMEM((128, D), jnp.float32),
            pltpu.SemaphoreType.DMA,
        ],
    )(m, src2d, dst2d)


def _sum_body(u_hbm, v_hbm, src2d, dst2d, s_out,
              sidx_v, didx_v, U_v, V_v, S_v, sem):
    w = _worker()
    row0 = w * ROWS_PW

    @pl.loop(0, NSUPER)
    def _bloop(b):
        pltpu.sync_copy(src2d.at[pl.ds(row0 + b * 16, 16)], sidx_v)
        pltpu.sync_copy(dst2d.at[pl.ds(row0 + b * 16, 16)], didx_v)

        @pl.loop(0, 16)
        def _jloop(j):
            pltpu.async_copy(u_hbm.at[sidx_v.at[j]], U_v, sem).wait()
            pltpu.async_copy(v_hbm.at[didx_v.at[j]], V_v, sem).wait()

            @pl.loop(0, 128)
            def _eloop(e):
                for k in range(D // 16):
                    S_v[e, pl.ds(16 * k, 16)] = jnp.maximum(
                        U_v[e, pl.ds(16 * k, 16)]
                        + V_v[e, pl.ds(16 * k, 16)], 0.0)
            pltpu.sync_copy(
                S_v, s_out.at[pl.ds((row0 + b * 16 + j) * 128, 128)])


def _sum_call(u, v, src2d, dst2d):
    return pl.kernel(
        _sum_body,
        out_type=jax.ShapeDtypeStruct((EPAD, D), jnp.float32),
        mesh=plsc.VectorSubcoreMesh(**_MESH),
        scratch_types=[
            pltpu.VMEM((16, 128), jnp.int32),
            pltpu.VMEM((16, 128), jnp.int32),
            pltpu.VMEM((128, D), jnp.float32),
            pltpu.VMEM((128, D), jnp.float32),
            pltpu.VMEM((128, D), jnp.float32),
            pltpu.SemaphoreType.DMA,
        ],
    )(u, v, src2d, dst2d)


MBLK = 8192


def _mlp_body(s_ref, w2_ref, b2_ref, o_ref):
    o_ref[...] = jax.nn.sigmoid(
        jnp.dot(s_ref[...], w2_ref[...], preferred_element_type=jnp.float32)
        + b2_ref[...])


def _mlp_call(s, w2, b2):
    return pl.pallas_call(
        _mlp_body,
        grid=(EPAD // MBLK,),
        in_specs=[
            pl.BlockSpec((MBLK, D), lambda i: (i, 0)),
            pl.BlockSpec((D, 1), lambda i: (0, 0)),
            pl.BlockSpec((1, 1), lambda i: (0, 0)),
        ],
        out_specs=pl.BlockSpec((MBLK, 1), lambda i: (i, 0)),
        out_shape=jax.ShapeDtypeStruct((EPAD, 1), jnp.float32),
    )(s, w2, b2)


def _cls_body(u_hbm, v_hbm, src2d, dst2d, w2_hbm, b2_hbm, out,
              sidx_v, didx_v, U_v, V_v, w2_v, b2_v, res_v, sem):
    pltpu.sync_copy(w2_hbm, w2_v)
    pltpu.sync_copy(b2_hbm, b2_v)
    w = _worker()
    row0 = w * ROWS_PW

    @pl.loop(0, NSUPER)
    def _bloop(b):
        pltpu.sync_copy(src2d.at[pl.ds(row0 + b * 16, 16)], sidx_v)
        pltpu.sync_copy(dst2d.at[pl.ds(row0 + b * 16, 16)], didx_v)

        @pl.loop(0, 16)
        def _jloop(j):
            pltpu.async_copy(u_hbm.at[sidx_v.at[j]], U_v, sem).wait()
            pltpu.async_copy(v_hbm.at[didx_v.at[j]], V_v, sem).wait()
            bv = b2_v[...]
            for g in range(8):
                rows = jnp.arange(16, dtype=jnp.int32) + g * 16

                def kkbody(kk, acc):
                    wch = w2_v[pl.ds(kk * 16, 16)]
                    for t in range(16):
                        cols = jnp.full((16,), kk * 16 + t, jnp.int32)
                        uu = plsc.load_gather(U_v, [rows, cols])
                        vv = plsc.load_gather(V_v, [rows, cols])
                        tt = jnp.maximum(uu + vv, 0.0)
                        acc = acc + tt * wch[t]
                    return acc
                acc = lax.fori_loop(0, 8, kkbody,
                                    jnp.zeros((16,), jnp.float32))
                res_v[pl.ds(g * 16, 16)] = 1.0 / (
                    1.0 + jnp.exp(-(acc + bv)))
            pltpu.sync_copy(
                res_v, out.at[pl.ds((row0 + b * 16 + j) * 128, 128)])


def _cls_call(u, v, src2d, dst2d, w2, b2):
    return pl.kernel(
        _cls_body,
        out_type=jax.ShapeDtypeStruct((EPAD,), jnp.float32),
        mesh=plsc.VectorSubcoreMesh(**_MESH),
        compiler_params=pltpu.CompilerParams(needs_layout_passes=False),
        scratch_types=[
            pltpu.VMEM((16, 128), jnp.int32),
            pltpu.VMEM((16, 128), jnp.int32),
            pltpu.VMEM((128, D), jnp.float32),
            pltpu.VMEM((128, D), jnp.float32),
            pltpu.VMEM((D,), jnp.float32),
            pltpu.VMEM((16,), jnp.float32),
            pltpu.VMEM((128,), jnp.float32),
            pltpu.SemaphoreType.DMA,
        ],
    )(u, v, src2d, dst2d, w2, b2)


BLK = 1280


def _prep_body(deg_ref, x_ref, w_ref, dinv_ref, m_ref):
    deg = deg_ref[0] + deg_ref[1] + 1.0
    dinv = lax.rsqrt(deg)
    h = jnp.dot(x_ref[...], w_ref[...], preferred_element_type=jnp.float32)
    dinv_ref[...] = dinv
    m_ref[...] = h * dinv


def _prep_call(degs, x, w):
    return pl.pallas_call(
        _prep_body,
        grid=(NPAD // BLK,),
        in_specs=[
            pl.BlockSpec((NC, BLK, 1), lambda i: (0, i, 0)),
            pl.BlockSpec((BLK, D), lambda i: (i, 0)),
            pl.BlockSpec((D, D), lambda i: (0, 0)),
        ],
        out_specs=[
            pl.BlockSpec((BLK, 1), lambda i: (i, 0)),
            pl.BlockSpec((BLK, D), lambda i: (i, 0)),
        ],
        out_shape=[
            jax.ShapeDtypeStruct((NPAD, 1), jnp.float32),
            jax.ShapeDtypeStruct((NPAD, D), jnp.float32),
        ],
    )(degs, x, w)


def _layer_body(acc_ref, mprev_ref, dinv_ref, b_ref, w_ref, mnext_ref):
    dinv = dinv_ref[...]
    z = jnp.maximum(
        dinv * (acc_ref[0] + acc_ref[1] - mprev_ref[...]) + b_ref[...], 0.0)
    mnext_ref[...] = jnp.dot(
        z, w_ref[...], preferred_element_type=jnp.float32) * dinv


def _layer_call(acc, mprev, dinv, b, w):
    return pl.pallas_call(
        _layer_body,
        grid=(NPAD // BLK,),
        in_specs=[
            pl.BlockSpec((NC, BLK, D), lambda i: (0, i, 0)),
            pl.BlockSpec((BLK, D), lambda i: (i, 0)),
            pl.BlockSpec((BLK, 1), lambda i: (i, 0)),
            pl.BlockSpec((1, D), lambda i: (0, 0)),
            pl.BlockSpec((D, D), lambda i: (0, 0)),
        ],
        out_specs=pl.BlockSpec((BLK, D), lambda i: (i, 0)),
        out_shape=jax.ShapeDtypeStruct((NPAD, D), jnp.float32),
    )(acc, mprev, dinv, b, w)


def _final_body(acc_ref, mprev_ref, dinv_ref, b_ref, wa_ref, wb_ref, bl1_ref,
                u_ref, v_ref):
    dinv = dinv_ref[...]
    z = jnp.maximum(
        dinv * (acc_ref[0] + acc_ref[1] - mprev_ref[...]) + b_ref[...], 0.0)
    u_ref[...] = jnp.dot(
        z, wa_ref[...], preferred_element_type=jnp.float32) + bl1_ref[...]
    v_ref[...] = jnp.dot(z, wb_ref[...], preferred_element_type=jnp.float32)


def _final_call(acc, mprev, dinv, b, wa, wb, bl1):
    return pl.pallas_call(
        _final_body,
        grid=(NPAD // BLK,),
        in_specs=[
            pl.BlockSpec((NC, BLK, D), lambda i: (0, i, 0)),
            pl.BlockSpec((BLK, D), lambda i: (i, 0)),
            pl.BlockSpec((BLK, 1), lambda i: (i, 0)),
            pl.BlockSpec((1, D), lambda i: (0, 0)),
            pl.BlockSpec((D, D), lambda i: (0, 0)),
            pl.BlockSpec((D, D), lambda i: (0, 0)),
            pl.BlockSpec((1, D), lambda i: (0, 0)),
        ],
        out_specs=[
            pl.BlockSpec((BLK, D), lambda i: (i, 0)),
            pl.BlockSpec((BLK, D), lambda i: (i, 0)),
        ],
        out_shape=[
            jax.ShapeDtypeStruct((NPAD, D), jnp.float32),
            jax.ShapeDtypeStruct((NPAD, D), jnp.float32),
        ],
    )(acc, mprev, dinv, b, wa, wb, bl1)


def kernel(x, edge_index, Wc0, bc0, Wc1, bc1, Wc2, bc2, Wl1, bl1, Wl2, bl2):
    ei = edge_index.astype(jnp.int32)
    pad = N + (jnp.arange(EPAD - E, dtype=jnp.int32) % (NPAD - N))
    src2d = jnp.concatenate([ei[0], pad]).reshape(IDXROWS, 128)
    dst2d = jnp.concatenate([ei[1], pad]).reshape(IDXROWS, 128)
    x_p = jnp.zeros((NPAD, D), jnp.float32).at[:N].set(x)

    degs = _deg_call(dst2d)
    dinv, m0 = _prep_call(degs.reshape(NC, NPAD, 1), x_p, Wc0)
    a0 = _agg_call(m0, src2d, dst2d)
    m1 = _layer_call(a0, m0, dinv, bc0.reshape(1, D), Wc1)
    a1 = _agg_call(m1, src2d, dst2d)
    m2 = _layer_call(a1, m1, dinv, bc1.reshape(1, D), Wc2)
    a2 = _agg_call(m2, src2d, dst2d)
    u, v = _final_call(a2, m2, dinv, bc2.reshape(1, D),
                       Wl1[:D], Wl1[D:], bl1.reshape(1, D))
    s = _sum_call(u, v, src2d, dst2d)
    outv = _mlp_call(s, Wl2, bl2.reshape(1, 1))
    return outv[:E]


# speedup vs baseline: 11.9031x; 1.1676x over previous
import functools

import jax
import jax.numpy as jnp
from jax import lax
from jax.experimental import pallas as pl
from jax.experimental.pallas import tpu as pltpu
from jax.experimental.pallas import tpu_sc as plsc

N = 10000
D = 128
E = 320000
NC, NS, LANES = 2, 16, 16
NW = NC * NS
NPAD = 10240
EPAD = 327680
ROWS_T = NPAD // NS
EPW = EPAD // NW
IDXROWS = EPAD // 128
ROWS_PW = EPW // 128
NSUPER = ROWS_PW // 16

_MESH = dict(core_axis_name="c", subcore_axis_name="s",
             num_cores=NC, num_subcores=NS)


def _worker():
    return lax.axis_index("c") * NS + lax.axis_index("s")


def _deg_body(dst2d, out, deg_sh, idx_v, ones_v, zero_v):
    c = lax.axis_index("c")
    s = lax.axis_index("s")

    @pl.loop(0, 128 // 16)
    def _fill(i):
        ones_v[pl.ds(i * 16, 16)] = jnp.full((16,), 1.0, jnp.float32)

    @pl.loop(0, ROWS_T // 16)
    def _zfill(i):
        zero_v[pl.ds(i * 16, 16)] = jnp.zeros((16,), jnp.float32)

    pltpu.sync_copy(zero_v, deg_sh.at[pl.ds(ROWS_T * s, ROWS_T)])
    plsc.subcore_barrier()

    w = _worker()
    row0 = w * ROWS_PW

    @pl.loop(0, NSUPER)
    def _bloop(b):
        pltpu.sync_copy(dst2d.at[pl.ds(row0 + b * 16, 16)], idx_v)

        @pl.loop(0, 16)
        def _jloop(j):
            pltpu.sync_copy(ones_v, deg_sh.at[idx_v.at[j]], add=True)

    plsc.subcore_barrier()
    pltpu.sync_copy(deg_sh.at[pl.ds(ROWS_T * s, ROWS_T)],
                    out.at[c, pl.ds(ROWS_T * s, ROWS_T)])


def _deg_call(dst2d):
    return pl.kernel(
        _deg_body,
        out_type=jax.ShapeDtypeStruct((NC, NPAD), jnp.float32),
        mesh=plsc.VectorSubcoreMesh(**_MESH),
        scratch_types=[
            pltpu.VMEM_SHARED((NPAD,), jnp.float32),
            pltpu.VMEM((16, 128), jnp.int32),
            pltpu.VMEM((128,), jnp.float32),
            pltpu.VMEM((ROWS_T,), jnp.float32),
        ],
    )(dst2d)


def _agg_body(m_hbm, src2d, dst2d, out, acc_sh, sidx_v, didx_v,
              rows_v, rows2_v, sem, sem2, ssem1, ssem2):
    c = lax.axis_index("c")
    s = lax.axis_index("s")
    pltpu.sync_copy(m_hbm.at[pl.ds(ROWS_T * s, ROWS_T)],
                    acc_sh.at[pl.ds(ROWS_T * s, ROWS_T)])
    plsc.subcore_barrier()

    w = _worker()
    row0 = w * ROWS_PW

    @pl.loop(0, NSUPER)
    def _bloop(b):
        pltpu.sync_copy(src2d.at[pl.ds(row0 + b * 16, 16)], sidx_v)
        pltpu.sync_copy(dst2d.at[pl.ds(row0 + b * 16, 16)], didx_v)
        rows = (rows_v, rows2_v)
        gsem = (sem, sem2)
        ssem = (ssem1, ssem2)
        gd = [None, None]
        sd = [None, None]
        gd[0] = pltpu.async_copy(m_hbm.at[sidx_v.at[0]], rows[0], gsem[0])
        for j in range(16):
            p = j & 1
            q = 1 - p
            if j + 1 < 16:
                if j >= 1:
                    sd[q].wait()
                gd[q] = pltpu.async_copy(
                    m_hbm.at[sidx_v.at[j + 1]], rows[q], gsem[q])
            gd[p].wait()
            sd[p] = pltpu.async_copy(
                rows[p], acc_sh.at[didx_v.at[j]], ssem[p], add=True)
        sd[0].wait()
        sd[1].wait()

    plsc.subcore_barrier()
    pltpu.sync_copy(acc_sh.at[pl.ds(ROWS_T * s, ROWS_T)],
                    out.at[c, pl.ds(ROWS_T * s, ROWS_T)])


def _agg_call(m, src2d, dst2d):
    return pl.kernel(
        _agg_body,
        out_type=jax.ShapeDtypeStruct((NC, NPAD, D), jnp.float32),
        mesh=plsc.VectorSubcoreMesh(**_MESH),
        scratch_types=[
            pltpu.VMEM_SHARED((NPAD, D), jnp.float32),
            pltpu.VMEM((16, 128), jnp.int32),
            pltpu.VMEM((16, 128), jnp.int32),
            pltpu.VMEM((128, D), jnp.float32),
            pltpu.VMEM((128, D), jnp.float32),
            pltpu.SemaphoreType.DMA,
            pltpu.SemaphoreType.DMA,
            pltpu.SemaphoreType.DMA,
            pltpu.SemaphoreType.DMA,
        ],
    )(m, src2d, dst2d)


def _sum_body(u_hbm, v_hbm, src2d, dst2d, s_out,
              sidx_v, didx_v, U_v, V_v, S_v, sem):
    w = _worker()
    row0 = w * ROWS_PW

    @pl.loop(0, NSUPER)
    def _bloop(b):
        pltpu.sync_copy(src2d.at[pl.ds(row0 + b * 16, 16)], sidx_v)
        pltpu.sync_copy(dst2d.at[pl.ds(row0 + b * 16, 16)], didx_v)

        @pl.loop(0, 16)
        def _jloop(j):
            pltpu.async_copy(u_hbm.at[sidx_v.at[j]], U_v, sem).wait()
            pltpu.async_copy(v_hbm.at[didx_v.at[j]], V_v, sem).wait()

            @pl.loop(0, 128)
            def _eloop(e):
                for k in range(D // 16):
                    S_v[e, pl.ds(16 * k, 16)] = jnp.maximum(
                        U_v[e, pl.ds(16 * k, 16)]
                        + V_v[e, pl.ds(16 * k, 16)], 0.0)
            pltpu.sync_copy(
                S_v, s_out.at[pl.ds((row0 + b * 16 + j) * 128, 128)])


def _sum_call(u, v, src2d, dst2d):
    return pl.kernel(
        _sum_body,
        out_type=jax.ShapeDtypeStruct((EPAD, D), jnp.float32),
        mesh=plsc.VectorSubcoreMesh(**_MESH),
        scratch_types=[
            pltpu.VMEM((16, 128), jnp.int32),
            pltpu.VMEM((16, 128), jnp.int32),
            pltpu.VMEM((128, D), jnp.float32),
            pltpu.VMEM((128, D), jnp.float32),
            pltpu.VMEM((128, D), jnp.float32),
            pltpu.SemaphoreType.DMA,
        ],
    )(u, v, src2d, dst2d)


MBLK = 8192


def _mlp_body(s_ref, w2_ref, b2_ref, o_ref):
    o_ref[...] = jax.nn.sigmoid(
        jnp.dot(s_ref[...], w2_ref[...], preferred_element_type=jnp.float32)
        + b2_ref[...])


def _mlp_call(s, w2, b2):
    return pl.pallas_call(
        _mlp_body,
        grid=(EPAD // MBLK,),
        in_specs=[
            pl.BlockSpec((MBLK, D), lambda i: (i, 0)),
            pl.BlockSpec((D, 1), lambda i: (0, 0)),
            pl.BlockSpec((1, 1), lambda i: (0, 0)),
        ],
        out_specs=pl.BlockSpec((MBLK, 1), lambda i: (i, 0)),
        out_shape=jax.ShapeDtypeStruct((EPAD, 1), jnp.float32),
    )(s, w2, b2)


def _cls_body(u_hbm, v_hbm, src2d, dst2d, w2_hbm, b2_hbm, out,
              sidx_v, didx_v, U_v, V_v, w2_v, b2_v, res_v, sem):
    pltpu.sync_copy(w2_hbm, w2_v)
    pltpu.sync_copy(b2_hbm, b2_v)
    w = _worker()
    row0 = w * ROWS_PW

    @pl.loop(0, NSUPER)
    def _bloop(b):
        pltpu.sync_copy(src2d.at[pl.ds(row0 + b * 16, 16)], sidx_v)
        pltpu.sync_copy(dst2d.at[pl.ds(row0 + b * 16, 16)], didx_v)

        @pl.loop(0, 16)
        def _jloop(j):
            pltpu.async_copy(u_hbm.at[sidx_v.at[j]], U_v, sem).wait()
            pltpu.async_copy(v_hbm.at[didx_v.at[j]], V_v, sem).wait()
            bv = b2_v[...]
            for g in range(8):
                rows = jnp.arange(16, dtype=jnp.int32) + g * 16

                def kkbody(kk, acc):
                    wch = w2_v[pl.ds(kk * 16, 16)]
                    for t in range(16):
                        cols = jnp.full((16,), kk * 16 + t, jnp.int32)
                        uu = plsc.load_gather(U_v, [rows, cols])
                        vv = plsc.load_gather(V_v, [rows, cols])
                        tt = jnp.maximum(uu + vv, 0.0)
                        acc = acc + tt * wch[t]
                    return acc
                acc = lax.fori_loop(0, 8, kkbody,
                                    jnp.zeros((16,), jnp.float32))
                res_v[pl.ds(g * 16, 16)] = 1.0 / (
                    1.0 + jnp.exp(-(acc + bv)))
            pltpu.sync_copy(
                res_v, out.at[pl.ds((row0 + b * 16 + j) * 128, 128)])


def _cls_call(u, v, src2d, dst2d, w2, b2):
    return pl.kernel(
        _cls_body,
        out_type=jax.ShapeDtypeStruct((EPAD,), jnp.float32),
        mesh=plsc.VectorSubcoreMesh(**_MESH),
        compiler_params=pltpu.CompilerParams(needs_layout_passes=False),
        scratch_types=[
            pltpu.VMEM((16, 128), jnp.int32),
            pltpu.VMEM((16, 128), jnp.int32),
            pltpu.VMEM((128, D), jnp.float32),
            pltpu.VMEM((128, D), jnp.float32),
            pltpu.VMEM((D,), jnp.float32),
            pltpu.VMEM((16,), jnp.float32),
            pltpu.VMEM((128,), jnp.float32),
            pltpu.SemaphoreType.DMA,
        ],
    )(u, v, src2d, dst2d, w2, b2)


BLK = 1280


def _prep_body(deg_ref, x_ref, w_ref, dinv_ref, m_ref):
    deg = deg_ref[0] + deg_ref[1] + 1.0
    dinv = lax.rsqrt(deg)
    h = jnp.dot(x_ref[...], w_ref[...], preferred_element_type=jnp.float32)
    dinv_ref[...] = dinv
    m_ref[...] = h * dinv


def _prep_call(degs, x, w):
    return pl.pallas_call(
        _prep_body,
        grid=(NPAD // BLK,),
        in_specs=[
            pl.BlockSpec((NC, BLK, 1), lambda i: (0, i, 0)),
            pl.BlockSpec((BLK, D), lambda i: (i, 0)),
            pl.BlockSpec((D, D), lambda i: (0, 0)),
        ],
        out_specs=[
            pl.BlockSpec((BLK, 1), lambda i: (i, 0)),
            pl.BlockSpec((BLK, D), lambda i: (i, 0)),
        ],
        out_shape=[
            jax.ShapeDtypeStruct((NPAD, 1), jnp.float32),
            jax.ShapeDtypeStruct((NPAD, D), jnp.float32),
        ],
    )(degs, x, w)


def _layer_body(acc_ref, mprev_ref, dinv_ref, b_ref, w_ref, mnext_ref):
    dinv = dinv_ref[...]
    z = jnp.maximum(
        dinv * (acc_ref[0] + acc_ref[1] - mprev_ref[...]) + b_ref[...], 0.0)
    mnext_ref[...] = jnp.dot(
        z, w_ref[...], preferred_element_type=jnp.float32) * dinv


def _layer_call(acc, mprev, dinv, b, w):
    return pl.pallas_call(
        _layer_body,
        grid=(NPAD // BLK,),
        in_specs=[
            pl.BlockSpec((NC, BLK, D), lambda i: (0, i, 0)),
            pl.BlockSpec((BLK, D), lambda i: (i, 0)),
            pl.BlockSpec((BLK, 1), lambda i: (i, 0)),
            pl.BlockSpec((1, D), lambda i: (0, 0)),
            pl.BlockSpec((D, D), lambda i: (0, 0)),
        ],
        out_specs=pl.BlockSpec((BLK, D), lambda i: (i, 0)),
        out_shape=jax.ShapeDtypeStruct((NPAD, D), jnp.float32),
    )(acc, mprev, dinv, b, w)


def _final_body(acc_ref, mprev_ref, dinv_ref, b_ref, wa_ref, wb_ref, bl1_ref,
                u_ref, v_ref):
    dinv = dinv_ref[...]
    z = jnp.maximum(
        dinv * (acc_ref[0] + acc_ref[1] - mprev_ref[...]) + b_ref[...], 0.0)
    u_ref[...] = jnp.dot(
        z, wa_ref[...], preferred_element_type=jnp.float32) + bl1_ref[...]
    v_ref[...] = jnp.dot(z, wb_ref[...], preferred_element_type=jnp.float32)


def _final_call(acc, mprev, dinv, b, wa, wb, bl1):
    return pl.pallas_call(
        _final_body,
        grid=(NPAD // BLK,),
        in_specs=[
            pl.BlockSpec((NC, BLK, D), lambda i: (0, i, 0)),
            pl.BlockSpec((BLK, D), lambda i: (i, 0)),
            pl.BlockSpec((BLK, 1), lambda i: (i, 0)),
            pl.BlockSpec((1, D), lambda i: (0, 0)),
            pl.BlockSpec((D, D), lambda i: (0, 0)),
            pl.BlockSpec((D, D), lambda i: (0, 0)),
            pl.BlockSpec((1, D), lambda i: (0, 0)),
        ],
        out_specs=[
            pl.BlockSpec((BLK, D), lambda i: (i, 0)),
            pl.BlockSpec((BLK, D), lambda i: (i, 0)),
        ],
        out_shape=[
            jax.ShapeDtypeStruct((NPAD, D), jnp.float32),
            jax.ShapeDtypeStruct((NPAD, D), jnp.float32),
        ],
    )(acc, mprev, dinv, b, wa, wb, bl1)


def kernel(x, edge_index, Wc0, bc0, Wc1, bc1, Wc2, bc2, Wl1, bl1, Wl2, bl2):
    ei = edge_index.astype(jnp.int32)
    pad = N + (jnp.arange(EPAD - E, dtype=jnp.int32) % (NPAD - N))
    src2d = jnp.concatenate([ei[0], pad]).reshape(IDXROWS, 128)
    dst2d = jnp.concatenate([ei[1], pad]).reshape(IDXROWS, 128)
    x_p = jnp.zeros((NPAD, D), jnp.float32).at[:N].set(x)

    degs = _deg_call(dst2d)
    dinv, m0 = _prep_call(degs.reshape(NC, NPAD, 1), x_p, Wc0)
    a0 = _agg_call(m0, src2d, dst2d)
    m1 = _layer_call(a0, m0, dinv, bc0.reshape(1, D), Wc1)
    a1 = _agg_call(m1, src2d, dst2d)
    m2 = _layer_call(a1, m1, dinv, bc1.reshape(1, D), Wc2)
    a2 = _agg_call(m2, src2d, dst2d)
    u, v = _final_call(a2, m2, dinv, bc2.reshape(1, D),
                       Wl1[:D], Wl1[D:], bl1.reshape(1, D))
    s = _sum_call(u, v, src2d, dst2d)
    outv = _mlp_call(s, Wl2, bl2.reshape(1, 1))
    return outv[:E]


# speedup vs baseline: 14.1463x; 1.1885x over previous
import functools

import jax
import jax.numpy as jnp
from jax import lax
from jax.experimental import pallas as pl
from jax.experimental.pallas import tpu as pltpu
from jax.experimental.pallas import tpu_sc as plsc

N = 10000
D = 128
E = 320000
NC, NS, LANES = 2, 16, 16
NW = NC * NS
NPAD = 10240
EPAD = 327680
ROWS_T = NPAD // NS
EPW = EPAD // NW
IDXROWS = EPAD // 128
ROWS_PW = EPW // 128
NSUPER = ROWS_PW // 16

_MESH = dict(core_axis_name="c", subcore_axis_name="s",
             num_cores=NC, num_subcores=NS)


def _worker():
    return lax.axis_index("c") * NS + lax.axis_index("s")


def _deg_body(dst2d, out, deg_sh, idx_v, ones_v, zero_v):
    c = lax.axis_index("c")
    s = lax.axis_index("s")

    @pl.loop(0, 128 // 16)
    def _fill(i):
        ones_v[pl.ds(i * 16, 16)] = jnp.full((16,), 1.0, jnp.float32)

    @pl.loop(0, ROWS_T // 16)
    def _zfill(i):
        zero_v[pl.ds(i * 16, 16)] = jnp.zeros((16,), jnp.float32)

    pltpu.sync_copy(zero_v, deg_sh.at[pl.ds(ROWS_T * s, ROWS_T)])
    plsc.subcore_barrier()

    w = _worker()
    row0 = w * ROWS_PW

    @pl.loop(0, NSUPER)
    def _bloop(b):
        pltpu.sync_copy(dst2d.at[pl.ds(row0 + b * 16, 16)], idx_v)

        @pl.loop(0, 16)
        def _jloop(j):
            pltpu.sync_copy(ones_v, deg_sh.at[idx_v.at[j]], add=True)

    plsc.subcore_barrier()
    pltpu.sync_copy(deg_sh.at[pl.ds(ROWS_T * s, ROWS_T)],
                    out.at[c, pl.ds(ROWS_T * s, ROWS_T)])


def _deg_call(dst2d):
    return pl.kernel(
        _deg_body,
        out_type=jax.ShapeDtypeStruct((NC, NPAD), jnp.float32),
        mesh=plsc.VectorSubcoreMesh(**_MESH),
        scratch_types=[
            pltpu.VMEM_SHARED((NPAD,), jnp.float32),
            pltpu.VMEM((16, 128), jnp.int32),
            pltpu.VMEM((128,), jnp.float32),
            pltpu.VMEM((ROWS_T,), jnp.float32),
        ],
    )(dst2d)


def _agg_body(m_hbm, src2d, dst2d, out, acc_sh, sidx_v, didx_v,
              rows_v, rows2_v, sem, sem2, ssem1, ssem2):
    c = lax.axis_index("c")
    s = lax.axis_index("s")
    pltpu.sync_copy(m_hbm.at[pl.ds(ROWS_T * s, ROWS_T)],
                    acc_sh.at[pl.ds(ROWS_T * s, ROWS_T)])
    plsc.subcore_barrier()

    w = _worker()
    row0 = w * ROWS_PW

    @pl.loop(0, NSUPER)
    def _bloop(b):
        pltpu.sync_copy(src2d.at[pl.ds(row0 + b * 16, 16)], sidx_v)
        pltpu.sync_copy(dst2d.at[pl.ds(row0 + b * 16, 16)], didx_v)
        rows = (rows_v, rows2_v)
        gsem = (sem, sem2)
        ssem = (ssem1, ssem2)
        gd = [None, None]
        sd = [None, None]
        gd[0] = pltpu.async_copy(m_hbm.at[sidx_v.at[0]], rows[0], gsem[0])
        for j in range(16):
            p = j & 1
            q = 1 - p
            if j + 1 < 16:
                if j >= 1:
                    sd[q].wait()
                gd[q] = pltpu.async_copy(
                    m_hbm.at[sidx_v.at[j + 1]], rows[q], gsem[q])
            gd[p].wait()
            sd[p] = pltpu.async_copy(
                rows[p], acc_sh.at[didx_v.at[j]], ssem[p], add=True)
        sd[0].wait()
        sd[1].wait()

    plsc.subcore_barrier()
    pltpu.sync_copy(acc_sh.at[pl.ds(ROWS_T * s, ROWS_T)],
                    out.at[c, pl.ds(ROWS_T * s, ROWS_T)])


def _agg_call(m, src2d, dst2d):
    return pl.kernel(
        _agg_body,
        out_type=jax.ShapeDtypeStruct((NC, NPAD, D), jnp.float32),
        mesh=plsc.VectorSubcoreMesh(**_MESH),
        scratch_types=[
            pltpu.VMEM_SHARED((NPAD, D), jnp.float32),
            pltpu.VMEM((16, 128), jnp.int32),
            pltpu.VMEM((16, 128), jnp.int32),
            pltpu.VMEM((128, D), jnp.float32),
            pltpu.VMEM((128, D), jnp.float32),
            pltpu.SemaphoreType.DMA,
            pltpu.SemaphoreType.DMA,
            pltpu.SemaphoreType.DMA,
            pltpu.SemaphoreType.DMA,
        ],
    )(m, src2d, dst2d)


def _dot_body(u_hbm, v_hbm, src2d, dst2d, w2_hbm, out,
              sidx_v, didx_v, U0, U1, V0, V1, w2_v, res_v,
              semu0, semu1, semv0, semv1):
    pltpu.sync_copy(w2_hbm, w2_v)
    w = _worker()
    row0 = w * ROWS_PW

    @pl.loop(0, NSUPER)
    def _bloop(b):
        pltpu.sync_copy(src2d.at[pl.ds(row0 + b * 16, 16)], sidx_v)
        pltpu.sync_copy(dst2d.at[pl.ds(row0 + b * 16, 16)], didx_v)
        U = (U0, U1)
        V = (V0, V1)
        semu = (semu0, semu1)
        semv = (semv0, semv1)
        gu = [None, None]
        gv = [None, None]
        gu[0] = pltpu.async_copy(u_hbm.at[sidx_v.at[0]], U[0], semu[0])
        gv[0] = pltpu.async_copy(v_hbm.at[didx_v.at[0]], V[0], semv[0])
        w2c = [w2_v[pl.ds(16 * i, 16)] for i in range(D // 16)]
        for j in range(16):
            p = j & 1
            q = 1 - p
            if j + 1 < 16:
                gu[q] = pltpu.async_copy(
                    u_hbm.at[sidx_v.at[j + 1]], U[q], semu[q])
                gv[q] = pltpu.async_copy(
                    v_hbm.at[didx_v.at[j + 1]], V[q], semv[q])
            gu[p].wait()
            gv[p].wait()
            Uv = U[p]
            Vv = V[p]

            @pl.loop(0, 128)
            def _eloop(e):
                acc = jnp.zeros((16,), jnp.float32)
                for k in range(D // 16):
                    t = jnp.maximum(
                        Uv[e, pl.ds(16 * k, 16)]
                        + Vv[e, pl.ds(16 * k, 16)], 0.0)
                    acc = acc + t * w2c[k]
                res_v[e, :] = acc
            pltpu.sync_copy(
                res_v, out.at[pl.ds((row0 + b * 16 + j) * 128, 128)])


def _dot_call(u, v, src2d, dst2d, w2):
    return pl.kernel(
        _dot_body,
        out_type=jax.ShapeDtypeStruct((EPAD, 16), jnp.float32),
        mesh=plsc.VectorSubcoreMesh(**_MESH),
        scratch_types=[
            pltpu.VMEM((16, 128), jnp.int32),
            pltpu.VMEM((16, 128), jnp.int32),
            pltpu.VMEM((128, D), jnp.float32),
            pltpu.VMEM((128, D), jnp.float32),
            pltpu.VMEM((128, D), jnp.float32),
            pltpu.VMEM((128, D), jnp.float32),
            pltpu.VMEM((D,), jnp.float32),
            pltpu.VMEM((128, 16), jnp.float32),
            pltpu.SemaphoreType.DMA,
            pltpu.SemaphoreType.DMA,
            pltpu.SemaphoreType.DMA,
            pltpu.SemaphoreType.DMA,
        ],
    )(u, v, src2d, dst2d, w2)


MBLK = 16384


def _fin_body(p_ref, b2_ref, o_ref):
    o_ref[...] = jax.nn.sigmoid(
        jnp.sum(p_ref[...], axis=1, keepdims=True) + b2_ref[...])


def _fin_call(p, b2):
    return pl.pallas_call(
        _fin_body,
        grid=(EPAD // MBLK,),
        in_specs=[
            pl.BlockSpec((MBLK, 16), lambda i: (i, 0)),
            pl.BlockSpec((1, 1), lambda i: (0, 0)),
        ],
        out_specs=pl.BlockSpec((MBLK, 1), lambda i: (i, 0)),
        out_shape=jax.ShapeDtypeStruct((EPAD, 1), jnp.float32),
    )(p, b2)


BLK = 1280


def _prep_body(deg_ref, x_ref, w_ref, dinv_ref, m_ref):
    deg = deg_ref[0] + deg_ref[1] + 1.0
    dinv = lax.rsqrt(deg)
    h = jnp.dot(x_ref[...], w_ref[...], preferred_element_type=jnp.float32)
    dinv_ref[...] = dinv
    m_ref[...] = h * dinv


def _prep_call(degs, x, w):
    return pl.pallas_call(
        _prep_body,
        grid=(NPAD // BLK,),
        in_specs=[
            pl.BlockSpec((NC, BLK, 1), lambda i: (0, i, 0)),
            pl.BlockSpec((BLK, D), lambda i: (i, 0)),
            pl.BlockSpec((D, D), lambda i: (0, 0)),
        ],
        out_specs=[
            pl.BlockSpec((BLK, 1), lambda i: (i, 0)),
            pl.BlockSpec((BLK, D), lambda i: (i, 0)),
        ],
        out_shape=[
            jax.ShapeDtypeStruct((NPAD, 1), jnp.float32),
            jax.ShapeDtypeStruct((NPAD, D), jnp.float32),
        ],
    )(degs, x, w)


def _layer_body(acc_ref, mprev_ref, dinv_ref, b_ref, w_ref, mnext_ref):
    dinv = dinv_ref[...]
    z = jnp.maximum(
        dinv * (acc_ref[0] + acc_ref[1] - mprev_ref[...]) + b_ref[...], 0.0)
    mnext_ref[...] = jnp.dot(
        z, w_ref[...], preferred_element_type=jnp.float32) * dinv


def _layer_call(acc, mprev, dinv, b, w):
    return pl.pallas_call(
        _layer_body,
        grid=(NPAD // BLK,),
        in_specs=[
            pl.BlockSpec((NC, BLK, D), lambda i: (0, i, 0)),
            pl.BlockSpec((BLK, D), lambda i: (i, 0)),
            pl.BlockSpec((BLK, 1), lambda i: (i, 0)),
            pl.BlockSpec((1, D), lambda i: (0, 0)),
            pl.BlockSpec((D, D), lambda i: (0, 0)),
        ],
        out_specs=pl.BlockSpec((BLK, D), lambda i: (i, 0)),
        out_shape=jax.ShapeDtypeStruct((NPAD, D), jnp.float32),
    )(acc, mprev, dinv, b, w)


def _final_body(acc_ref, mprev_ref, dinv_ref, b_ref, wa_ref, wb_ref, bl1_ref,
                u_ref, v_ref):
    dinv = dinv_ref[...]
    z = jnp.maximum(
        dinv * (acc_ref[0] + acc_ref[1] - mprev_ref[...]) + b_ref[...], 0.0)
    u_ref[...] = jnp.dot(
        z, wa_ref[...], preferred_element_type=jnp.float32) + bl1_ref[...]
    v_ref[...] = jnp.dot(z, wb_ref[...], preferred_element_type=jnp.float32)


def _final_call(acc, mprev, dinv, b, wa, wb, bl1):
    return pl.pallas_call(
        _final_body,
        grid=(NPAD // BLK,),
        in_specs=[
            pl.BlockSpec((NC, BLK, D), lambda i: (0, i, 0)),
            pl.BlockSpec((BLK, D), lambda i: (i, 0)),
            pl.BlockSpec((BLK, 1), lambda i: (i, 0)),
            pl.BlockSpec((1, D), lambda i: (0, 0)),
            pl.BlockSpec((D, D), lambda i: (0, 0)),
            pl.BlockSpec((D, D), lambda i: (0, 0)),
            pl.BlockSpec((1, D), lambda i: (0, 0)),
        ],
        out_specs=[
            pl.BlockSpec((BLK, D), lambda i: (i, 0)),
            pl.BlockSpec((BLK, D), lambda i: (i, 0)),
        ],
        out_shape=[
            jax.ShapeDtypeStruct((NPAD, D), jnp.float32),
            jax.ShapeDtypeStruct((NPAD, D), jnp.float32),
        ],
    )(acc, mprev, dinv, b, wa, wb, bl1)


def kernel(x, edge_index, Wc0, bc0, Wc1, bc1, Wc2, bc2, Wl1, bl1, Wl2, bl2):
    ei = edge_index.astype(jnp.int32)
    pad = N + (jnp.arange(EPAD - E, dtype=jnp.int32) % (NPAD - N))
    src2d = jnp.concatenate([ei[0], pad]).reshape(IDXROWS, 128)
    dst2d = jnp.concatenate([ei[1], pad]).reshape(IDXROWS, 128)
    x_p = jnp.zeros((NPAD, D), jnp.float32).at[:N].set(x)

    degs = _deg_call(dst2d)
    dinv, m0 = _prep_call(degs.reshape(NC, NPAD, 1), x_p, Wc0)
    a0 = _agg_call(m0, src2d, dst2d)
    m1 = _layer_call(a0, m0, dinv, bc0.reshape(1, D), Wc1)
    a1 = _agg_call(m1, src2d, dst2d)
    m2 = _layer_call(a1, m1, dinv, bc1.reshape(1, D), Wc2)
    a2 = _agg_call(m2, src2d, dst2d)
    u, v = _final_call(a2, m2, dinv, bc2.reshape(1, D),
                       Wl1[:D], Wl1[D:], bl1.reshape(1, D))
    p = _dot_call(u, v, src2d, dst2d, Wl2.reshape(D))
    outv = _fin_call(p, bl2.reshape(1, 1))
    return outv[:E]
